# baseline (speedup 1.0000x reference)
import jax
import jax.numpy as jnp
from jax.experimental import pallas as pl
from jax.experimental.pallas import tpu as pltpu


def _cdfg_kernel(idx_ref, xs_ref, a0_ref, a1_ref, m_ref,
                 win_ref, bin_ref, w1_ref, b1_ref, w2_ref, b2_ref,
                 out_ref, x_scr):
    b = pl.program_id(0)
    nb = pl.num_programs(0)
    s = idx_ref[b, 1]
    m = m_ref[pl.ds(s, 1), :]
    prev = idx_ref[jnp.maximum(b - 1, 0), 0]
    dup = jnp.logical_and(b > 0, idx_ref[b, 0] == prev)
    nxt = idx_ref[jnp.minimum(b + 1, nb - 1), 0]
    next_dup = jnp.logical_and(b < nb - 1, nxt == idx_ref[b, 0])

    def finish(x):
        num = jnp.dot(m, x, preferred_element_type=jnp.float32)
        den = jnp.sum(m)
        out_ref[pl.ds(s, 1), :] = num / den

    @pl.when(jnp.logical_not(dup))
    def _compute():
        xs = xs_ref[0]

        def conv(y):
            return jnp.concatenate(
                [jnp.dot(p[0], y, preferred_element_type=jnp.float32)
                 for p in (a0_ref, a1_ref)], axis=0)

        x0 = jnp.maximum(
            jnp.dot(xs, win_ref[...], preferred_element_type=jnp.float32)
            + bin_ref[...], 0.0)
        y1 = jnp.dot(x0, w1_ref[...], preferred_element_type=jnp.float32)
        x1 = jnp.maximum(conv(y1) + b1_ref[...], 0.0)
        y2 = jnp.dot(x1, w2_ref[...], preferred_element_type=jnp.float32)
        x2 = jnp.tanh(conv(y2) + b2_ref[...])
        x = x2 + x0

        @pl.when(next_dup)
        def _save():
            x_scr[...] = x

        finish(x)

    @pl.when(dup)
    def _reuse():
        finish(x_scr[...])


def kernel(graph, coverpoint, coverpoint_mask, batch_xs, batch_as,
           W_in, b_in, W1, b1, W2, b2):
    B = graph.shape[0]
    _, N, F = batch_xs.shape
    H = W1.shape[1]
    NH = N // 2

    ids = graph[:, 0].astype(jnp.int32)
    order = jnp.argsort(ids)
    packed = jnp.stack([ids[order], order.astype(jnp.int32)], axis=1)

    grid_spec = pltpu.PrefetchScalarGridSpec(
        num_scalar_prefetch=1,
        grid=(B,),
        in_specs=[
            pl.BlockSpec((1, N, F), lambda b, i: (i[b, 0], 0, 0)),
            pl.BlockSpec((1, NH, N), lambda b, i: (i[b, 0], 0, 0)),
            pl.BlockSpec((1, NH, N), lambda b, i: (i[b, 0], 1, 0)),
            pl.BlockSpec((B, N), lambda b, i: (0, 0)),
            pl.BlockSpec((F, H), lambda b, i: (0, 0)),
            pl.BlockSpec((1, H), lambda b, i: (0, 0)),
            pl.BlockSpec((H, H), lambda b, i: (0, 0)),
            pl.BlockSpec((1, H), lambda b, i: (0, 0)),
            pl.BlockSpec((H, H), lambda b, i: (0, 0)),
            pl.BlockSpec((1, H), lambda b, i: (0, 0)),
        ],
        out_specs=pl.BlockSpec((B, H), lambda b, i: (0, 0)),
        scratch_shapes=[pltpu.VMEM((N, H), jnp.float32)],
    )
    return pl.pallas_call(
        _cdfg_kernel,
        grid_spec=grid_spec,
        out_shape=jax.ShapeDtypeStruct((B, H), jnp.float32),
        compiler_params=pltpu.CompilerParams(
            vmem_limit_bytes=100 * 1024 * 1024),
    )(packed, batch_xs, batch_as, batch_as,
      coverpoint_mask.astype(jnp.float32),
      W_in, b_in.reshape(1, -1), W1, b1.reshape(1, -1), W2, b2.reshape(1, -1))


# speedup vs baseline: 1.0398x; 1.0398x over previous
import jax
import jax.numpy as jnp
from jax.experimental import pallas as pl
from jax.experimental.pallas import tpu as pltpu


def _cdfg_kernel(idx_ref, xs_ref, a0_ref, a1_ref, m_ref,
                 win_ref, bin_ref, w1_ref, b1_ref, w2_ref, b2_ref,
                 out_ref, x_scr):
    b = pl.program_id(0)
    nb = pl.num_programs(0)
    s = idx_ref[b, 1]
    m = m_ref[pl.ds(s, 1), :]
    prev = idx_ref[jnp.maximum(b - 1, 0), 0]
    dup = jnp.logical_and(b > 0, idx_ref[b, 0] == prev)
    nxt = idx_ref[jnp.minimum(b + 1, nb - 1), 0]
    next_dup = jnp.logical_and(b < nb - 1, nxt == idx_ref[b, 0])

    def finish(x):
        num = jnp.dot(m, x, preferred_element_type=jnp.float32)
        den = jnp.sum(m)
        out_ref[pl.ds(s, 1), :] = num / den

    @pl.when(jnp.logical_not(dup))
    def _compute():
        xs = xs_ref[0]

        def conv(y):
            return jnp.concatenate(
                [jnp.dot(p[0], y, preferred_element_type=jnp.float32)
                 for p in (a0_ref, a1_ref)], axis=0)

        x0 = jnp.maximum(
            jnp.dot(xs, win_ref[...], preferred_element_type=jnp.float32)
            + bin_ref[...], 0.0)
        y1 = jnp.dot(x0, w1_ref[...], preferred_element_type=jnp.float32)
        x1 = jnp.maximum(conv(y1) + b1_ref[...], 0.0)
        y2 = jnp.dot(x1, w2_ref[...], preferred_element_type=jnp.float32)
        x2 = jnp.tanh(conv(y2) + b2_ref[...])
        x = x2 + x0

        @pl.when(next_dup)
        def _save():
            x_scr[...] = x

        finish(x)

    @pl.when(dup)
    def _reuse():
        finish(x_scr[...])


def kernel(graph, coverpoint, coverpoint_mask, batch_xs, batch_as,
           W_in, b_in, W1, b1, W2, b2):
    B = graph.shape[0]
    _, N, F = batch_xs.shape
    H = W1.shape[1]
    NH = N // 2

    ids = graph[:, 0].astype(jnp.int32)
    lt = (ids[:, None] > ids[None, :]).astype(jnp.int32)
    tie = jnp.logical_and(ids[:, None] == ids[None, :],
                          jnp.arange(B)[:, None] > jnp.arange(B)[None, :])
    rank = jnp.sum(lt + tie.astype(jnp.int32), axis=1)
    onehot = (rank[:, None] == jnp.arange(B)[None, :]).astype(jnp.int32)
    sorted_ids = jnp.sum(onehot * ids[:, None], axis=0)
    inv = jnp.sum(onehot * jnp.arange(B)[:, None], axis=0)
    packed = jnp.stack([sorted_ids, inv.astype(jnp.int32)], axis=1)

    grid_spec = pltpu.PrefetchScalarGridSpec(
        num_scalar_prefetch=1,
        grid=(B,),
        in_specs=[
            pl.BlockSpec((1, N, F), lambda b, i: (i[b, 0], 0, 0)),
            pl.BlockSpec((1, NH, N), lambda b, i: (i[b, 0], 0, 0)),
            pl.BlockSpec((1, NH, N), lambda b, i: (i[b, 0], 1, 0)),
            pl.BlockSpec((B, N), lambda b, i: (0, 0)),
            pl.BlockSpec((F, H), lambda b, i: (0, 0)),
            pl.BlockSpec((1, H), lambda b, i: (0, 0)),
            pl.BlockSpec((H, H), lambda b, i: (0, 0)),
            pl.BlockSpec((1, H), lambda b, i: (0, 0)),
            pl.BlockSpec((H, H), lambda b, i: (0, 0)),
            pl.BlockSpec((1, H), lambda b, i: (0, 0)),
        ],
        out_specs=pl.BlockSpec((B, H), lambda b, i: (0, 0)),
        scratch_shapes=[pltpu.VMEM((N, H), jnp.float32)],
    )
    return pl.pallas_call(
        _cdfg_kernel,
        grid_spec=grid_spec,
        out_shape=jax.ShapeDtypeStruct((B, H), jnp.float32),
        compiler_params=pltpu.CompilerParams(
            vmem_limit_bytes=100 * 1024 * 1024),
    )(packed, batch_xs, batch_as, batch_as,
      coverpoint_mask.astype(jnp.float32),
      W_in, b_in.reshape(1, -1), W1, b1.reshape(1, -1), W2, b2.reshape(1, -1))


# speedup vs baseline: 1.0637x; 1.0230x over previous
import jax
import jax.numpy as jnp
from jax.experimental import pallas as pl
from jax.experimental.pallas import tpu as pltpu


def _cdfg_kernel(idx_ref, xs_ref, a0_ref, a1_ref, m_ref,
                 win_ref, bin_ref, w1_ref, b1_ref, w2_ref, b2_ref, out_ref):
    b = pl.program_id(0)
    s = idx_ref[b, 1]
    xs = xs_ref[0]
    m = m_ref[pl.ds(s, 1), :]

    def conv(y):
        return jnp.concatenate(
            [jnp.dot(p[0], y, preferred_element_type=jnp.float32)
             for p in (a0_ref, a1_ref)], axis=0)

    x0 = jnp.maximum(
        jnp.dot(xs, win_ref[...], preferred_element_type=jnp.float32)
        + bin_ref[...], 0.0)
    y1 = jnp.dot(x0, w1_ref[...], preferred_element_type=jnp.float32)
    x1 = jnp.maximum(conv(y1) + b1_ref[...], 0.0)
    y2 = jnp.dot(x1, w2_ref[...], preferred_element_type=jnp.float32)
    x2 = jnp.tanh(conv(y2) + b2_ref[...])
    x = x2 + x0
    num = jnp.dot(m, x, preferred_element_type=jnp.float32)
    den = jnp.sum(m)
    out_ref[pl.ds(s, 1), :] = num / den


def kernel(graph, coverpoint, coverpoint_mask, batch_xs, batch_as,
           W_in, b_in, W1, b1, W2, b2):
    B = graph.shape[0]
    _, N, F = batch_xs.shape
    H = W1.shape[1]
    NH = N // 2

    ids = graph[:, 0].astype(jnp.int32)
    lt = (ids[:, None] > ids[None, :]).astype(jnp.int32)
    tie = jnp.logical_and(ids[:, None] == ids[None, :],
                          jnp.arange(B)[:, None] > jnp.arange(B)[None, :])
    rank = jnp.sum(lt + tie.astype(jnp.int32), axis=1)
    onehot = (rank[:, None] == jnp.arange(B)[None, :]).astype(jnp.int32)
    sorted_ids = jnp.sum(onehot * ids[:, None], axis=0)
    inv = jnp.sum(onehot * jnp.arange(B)[:, None], axis=0)
    packed = jnp.stack([sorted_ids, inv.astype(jnp.int32)], axis=1)

    grid_spec = pltpu.PrefetchScalarGridSpec(
        num_scalar_prefetch=1,
        grid=(B,),
        in_specs=[
            pl.BlockSpec((1, N, F), lambda b, i: (i[b, 0], 0, 0)),
            pl.BlockSpec((1, NH, N), lambda b, i: (i[b, 0], 0, 0)),
            pl.BlockSpec((1, NH, N), lambda b, i: (i[b, 0], 1, 0)),
            pl.BlockSpec((B, N), lambda b, i: (0, 0)),
            pl.BlockSpec((F, H), lambda b, i: (0, 0)),
            pl.BlockSpec((1, H), lambda b, i: (0, 0)),
            pl.BlockSpec((H, H), lambda b, i: (0, 0)),
            pl.BlockSpec((1, H), lambda b, i: (0, 0)),
            pl.BlockSpec((H, H), lambda b, i: (0, 0)),
            pl.BlockSpec((1, H), lambda b, i: (0, 0)),
        ],
        out_specs=pl.BlockSpec((B, H), lambda b, i: (0, 0)),
    )
    return pl.pallas_call(
        _cdfg_kernel,
        grid_spec=grid_spec,
        out_shape=jax.ShapeDtypeStruct((B, H), jnp.float32),
        compiler_params=pltpu.CompilerParams(
            vmem_limit_bytes=100 * 1024 * 1024),
    )(packed, batch_xs, batch_as, batch_as,
      coverpoint_mask.astype(jnp.float32),
      W_in, b_in.reshape(1, -1), W1, b1.reshape(1, -1), W2, b2.reshape(1, -1))
